# baseline (speedup 1.0000x reference)
import functools

import jax
import jax.numpy as jnp
from jax import lax
from jax.experimental import pallas as pl
from jax.experimental.pallas import tpu as pltpu
from jax.experimental.pallas import tpu_sc as plsc

N = 10000
E = 320000
D = 128
H = 128
NROWS = 10240
NC = 2
NS = 16
NW = NC * NS
CHUNK = 128
NCHUNK = -(-E // (NW * 2 * CHUNK)) * 2
EPW = NCHUNK * CHUNK
EPAD = EPW * NW
ROWS_PER_TILE = NROWS // NS
BLK = 1000


def _sc_aggregate(y, srcp, dstp):
    mesh = plsc.VectorSubcoreMesh(core_axis_name="c", subcore_axis_name="s")

    @functools.partial(
        pl.kernel,
        out_type=jax.ShapeDtypeStruct((NC, NROWS, H), jnp.float32),
        mesh=mesh,
        scratch_types=[
            pltpu.VMEM((NCHUNK // 2, CHUNK), jnp.int32),
            pltpu.VMEM((NCHUNK // 2, CHUNK), jnp.int32),
            pltpu.VMEM((2, CHUNK, H), jnp.float32),
            pltpu.VMEM_SHARED((NROWS, H), jnp.float32),
            pltpu.SemaphoreType.DMA,
            pltpu.SemaphoreType.DMA,
        ],
    )
    def body(y_hbm, src_hbm, dst_hbm, out_hbm, srci_v, dsti_v, rows_v, acc_sh,
             sem0, sem1):
        cid = lax.axis_index("c")
        sid = lax.axis_index("s")
        wid = sid * NC + cid
        HALF = NCHUNK // 2

        def zero_row(j, carry):
            for k in range(H // 16):
                rows_v[0, j, pl.ds(k * 16, 16)] = jnp.zeros((16,), jnp.float32)
            return carry
        lax.fori_loop(0, CHUNK, zero_row, 0)
        for r in range(ROWS_PER_TILE // CHUNK):
            pltpu.sync_copy(rows_v.at[0],
                            acc_sh.at[pl.ds(sid * ROWS_PER_TILE + r * CHUNK, CHUNK)])
        plsc.subcore_barrier()

        def issue(c, buf, sem):
            pltpu.async_copy(y_hbm.at[srci_v.at[c]], rows_v.at[buf], sem)

        def drain(c, buf, sem):
            pltpu.make_async_copy(y_hbm.at[srci_v.at[c]], rows_v.at[buf], sem).wait()
            pltpu.sync_copy(rows_v.at[buf], acc_sh.at[dsti_v.at[c]], add=True)

        for half in range(2):
            pltpu.sync_copy(src_hbm.at[wid, pl.ds(half * HALF, HALF)], srci_v)
            pltpu.sync_copy(dst_hbm.at[wid, pl.ds(half * HALF, HALF)], dsti_v)
            issue(0, 0, sem0)

            def pair_body(i, carry):
                c = 2 * i
                issue(c + 1, 1, sem1)
                drain(c, 0, sem0)
                issue(c + 2, 0, sem0)
                drain(c + 1, 1, sem1)
                return carry
            lax.fori_loop(0, HALF // 2 - 1, pair_body, 0)
            issue(HALF - 1, 1, sem1)
            drain(HALF - 2, 0, sem0)
            drain(HALF - 1, 1, sem1)
        plsc.subcore_barrier()

        pltpu.sync_copy(
            acc_sh.at[pl.ds(sid * ROWS_PER_TILE, ROWS_PER_TILE)],
            out_hbm.at[cid, pl.ds(sid * ROWS_PER_TILE, ROWS_PER_TILE)],
        )

    return body(y, srcp, dstp)


def _pre_body(x_ref, t_ref, W1x_ref, w1t_ref, P1x_ref, p1t_ref, y_ref, xcP_ref):
    x = x_ref[...]
    t = t_ref[...]
    y_ref[...] = (jnp.dot(x, W1x_ref[...], preferred_element_type=jnp.float32)
                  + t * w1t_ref[...])
    xcP_ref[...] = (jnp.dot(x, P1x_ref[...], preferred_element_type=jnp.float32)
                    + t * p1t_ref[...])


def _pre(x, t2, W1x, w1t, P1x, p1t):
    full = lambda shape: pl.BlockSpec(shape, lambda i: (0,) * len(shape))
    return pl.pallas_call(
        _pre_body,
        grid=(N // BLK,),
        in_specs=[
            pl.BlockSpec((BLK, D), lambda i: (i, 0)),
            pl.BlockSpec((BLK, 1), lambda i: (i, 0)),
            full((D, H)), full((1, H)), full((D, H)), full((1, H)),
        ],
        out_specs=[pl.BlockSpec((BLK, H), lambda i: (i, 0)),
                   pl.BlockSpec((BLK, H), lambda i: (i, 0))],
        out_shape=[jax.ShapeDtypeStruct((N, H), jnp.float32),
                   jax.ShapeDtypeStruct((N, H), jnp.float32)],
    )(x, t2, W1x, w1t, P1x, p1t)


def _post_body(y_ref, xcP_ref, parts_ref, b1_ref, W2_ref, b2_ref,
               P1h_ref, bp1_ref, P2_ref, bp2_ref, P3_ref, bp3_ref, out_ref):
    h = y_ref[...] + parts_ref[0] + parts_ref[1] + b1_ref[...]
    h = jnp.maximum(h, 0.0)
    h = jnp.tanh(jnp.dot(h, W2_ref[...], preferred_element_type=jnp.float32) + b2_ref[...])
    h = jnp.maximum(h, 0.0)
    p = (jnp.dot(h, P1h_ref[...], preferred_element_type=jnp.float32)
         + xcP_ref[...] + bp1_ref[...])
    p = jnp.where(p >= 0, p, 0.2 * p)
    p = jnp.dot(p, P2_ref[...], preferred_element_type=jnp.float32) + bp2_ref[...]
    p = jnp.where(p >= 0, p, 0.2 * p)
    out_ref[...] = jnp.sum(p * P3_ref[...], axis=1, keepdims=True) + bp3_ref[...]


def _post(y, xcP, parts, b1, W2, b2, P1h, bp1, P2, bp2, P3r, bp3):
    full = lambda shape: pl.BlockSpec(shape, lambda i: (0,) * len(shape))
    return pl.pallas_call(
        _post_body,
        grid=(N // BLK,),
        in_specs=[
            pl.BlockSpec((BLK, H), lambda i: (i, 0)),
            pl.BlockSpec((BLK, H), lambda i: (i, 0)),
            pl.BlockSpec((NC, BLK, H), lambda i: (0, i, 0)),
            full((1, H)), full((H, H)), full((1, H)),
            full((H, H)), full((1, H)), full((H, H)), full((1, H)),
            full((1, H)), full((1, 1)),
        ],
        out_specs=pl.BlockSpec((BLK, 1), lambda i: (i, 0)),
        out_shape=jax.ShapeDtypeStruct((N, 1), jnp.float32),
    )(y, xcP, parts, b1, W2, b2, P1h, bp1, P2, bp2, P3r, bp3)


def kernel(x, t, z, edge_index, W1, b1, W2, b2, P1, bp1, P2, bp2, P3, bp3):
    t2 = t[:, None]
    y, xcP = _pre(x, t2, W1[:D], W1[D:D + 1], P1[H:H + D], P1[H + D:H + D + 1])

    src = edge_index[0]
    dst = edge_index[1]
    pad = EPAD - E
    srcp = jnp.concatenate([src, jnp.zeros((pad,), jnp.int32)]).reshape(NW, NCHUNK, CHUNK)
    dstp = jnp.concatenate([dst, jnp.full((pad,), NROWS - 1, jnp.int32)]).reshape(NW, NCHUNK, CHUNK)

    parts = _sc_aggregate(y, srcp, dstp)

    p = _post(y, xcP, parts, b1[None, :], W2, b2[None, :],
              P1[:H], bp1[None, :], P2, bp2[None, :], P3.reshape(1, H), bp3[None, :])

    t_pred = jnp.zeros((N, 1), jnp.float32)
    return (t_pred, p)


# speedup vs baseline: 1.3631x; 1.3631x over previous
import functools

import jax
import jax.numpy as jnp
from jax import lax
from jax.experimental import pallas as pl
from jax.experimental.pallas import tpu as pltpu
from jax.experimental.pallas import tpu_sc as plsc

N = 10000
E = 320000
D = 128
H = 128
NROWS = 10240
NC = 2
NS = 16
NW = NC * NS
CHUNK = 128
C0 = 127
C1 = 30
E0 = NS * C0 * CHUNK
E1 = NS * C1 * CHUNK
EPAD = E0 + E1
ROWS_PER_TILE = NROWS // NS
BLK = 1000


def _sc_aggregate(y, src0, dst0, src1, dst1):
    mesh = plsc.VectorSubcoreMesh(core_axis_name="c", subcore_axis_name="s")

    @functools.partial(
        pl.kernel,
        out_type=jax.ShapeDtypeStruct((NC, NROWS, H), jnp.float32),
        mesh=mesh,
        scratch_types=[
            pltpu.VMEM((CHUNK,), jnp.int32),
            pltpu.VMEM((CHUNK,), jnp.int32),
            pltpu.VMEM((CHUNK, H), jnp.float32),
            pltpu.VMEM_SHARED((NROWS, H), jnp.float32),
            pltpu.SemaphoreType.DMA,
        ],
    )
    def body(y_hbm, src0_hbm, dst0_hbm, src1_hbm, dst1_hbm, out_hbm,
             srci_v, dsti_v, rows_v, acc_sh, sem):
        cid = lax.axis_index("c")
        sid = lax.axis_index("s")

        def zero_row(j, carry):
            for k in range(H // 16):
                rows_v[j, pl.ds(k * 16, 16)] = jnp.zeros((16,), jnp.float32)
            return carry
        lax.fori_loop(0, CHUNK, zero_row, 0)
        for r in range(ROWS_PER_TILE // CHUNK):
            pltpu.sync_copy(rows_v, acc_sh.at[pl.ds(sid * ROWS_PER_TILE + r * CHUNK, CHUNK)])
        plsc.subcore_barrier()

        def run_edges(src_hbm, dst_hbm, nchunk):
            def chunk_body(c, carry):
                pltpu.sync_copy(src_hbm.at[sid, c], srci_v)
                pltpu.sync_copy(dst_hbm.at[sid, c], dsti_v)
                pltpu.async_copy(y_hbm.at[srci_v], rows_v, sem).wait()
                pltpu.sync_copy(rows_v, acc_sh.at[dsti_v], add=True)
                return carry
            lax.fori_loop(0, nchunk, chunk_body, 0)

        @pl.when(cid == 0)
        def _():
            run_edges(src0_hbm, dst0_hbm, C0)

        @pl.when(cid == 1)
        def _():
            run_edges(src1_hbm, dst1_hbm, C1)
        plsc.subcore_barrier()

        pltpu.sync_copy(
            acc_sh.at[pl.ds(sid * ROWS_PER_TILE, ROWS_PER_TILE)],
            out_hbm.at[cid, pl.ds(sid * ROWS_PER_TILE, ROWS_PER_TILE)],
        )

    return body(y, src0, dst0, src1, dst1)


def _pre_body(x_ref, t_ref, W1x_ref, w1t_ref, P1x_ref, p1t_ref, y_ref, xcP_ref):
    x = x_ref[...]
    t = t_ref[...]
    y_ref[...] = (jnp.dot(x, W1x_ref[...], preferred_element_type=jnp.float32)
                  + t * w1t_ref[...])
    xcP_ref[...] = (jnp.dot(x, P1x_ref[...], preferred_element_type=jnp.float32)
                    + t * p1t_ref[...])


def _pre(x, t2, W1x, w1t, P1x, p1t):
    full = lambda shape: pl.BlockSpec(shape, lambda i: (0,) * len(shape))
    return pl.pallas_call(
        _pre_body,
        grid=(N // BLK,),
        in_specs=[
            pl.BlockSpec((BLK, D), lambda i: (i, 0)),
            pl.BlockSpec((BLK, 1), lambda i: (i, 0)),
            full((D, H)), full((1, H)), full((D, H)), full((1, H)),
        ],
        out_specs=[pl.BlockSpec((BLK, H), lambda i: (i, 0)),
                   pl.BlockSpec((BLK, H), lambda i: (i, 0))],
        out_shape=[jax.ShapeDtypeStruct((N, H), jnp.float32),
                   jax.ShapeDtypeStruct((N, H), jnp.float32)],
    )(x, t2, W1x, w1t, P1x, p1t)


def _post_body(y_ref, xcP_ref, parts_ref, b1_ref, W2_ref, b2_ref,
               P1h_ref, bp1_ref, P2_ref, bp2_ref, P3_ref, bp3_ref, out_ref):
    h = y_ref[...] + parts_ref[0] + parts_ref[1] + b1_ref[...]
    h = jnp.maximum(h, 0.0)
    h = jnp.tanh(jnp.dot(h, W2_ref[...], preferred_element_type=jnp.float32) + b2_ref[...])
    h = jnp.maximum(h, 0.0)
    p = (jnp.dot(h, P1h_ref[...], preferred_element_type=jnp.float32)
         + xcP_ref[...] + bp1_ref[...])
    p = jnp.where(p >= 0, p, 0.2 * p)
    p = jnp.dot(p, P2_ref[...], preferred_element_type=jnp.float32) + bp2_ref[...]
    p = jnp.where(p >= 0, p, 0.2 * p)
    out_ref[...] = jnp.sum(p * P3_ref[...], axis=1, keepdims=True) + bp3_ref[...]


def _post(y, xcP, parts, b1, W2, b2, P1h, bp1, P2, bp2, P3r, bp3):
    full = lambda shape: pl.BlockSpec(shape, lambda i: (0,) * len(shape))
    return pl.pallas_call(
        _post_body,
        grid=(N // BLK,),
        in_specs=[
            pl.BlockSpec((BLK, H), lambda i: (i, 0)),
            pl.BlockSpec((BLK, H), lambda i: (i, 0)),
            pl.BlockSpec((NC, BLK, H), lambda i: (0, i, 0)),
            full((1, H)), full((H, H)), full((1, H)),
            full((H, H)), full((1, H)), full((H, H)), full((1, H)),
            full((1, H)), full((1, 1)),
        ],
        out_specs=pl.BlockSpec((BLK, 1), lambda i: (i, 0)),
        out_shape=jax.ShapeDtypeStruct((N, 1), jnp.float32),
    )(y, xcP, parts, b1, W2, b2, P1h, bp1, P2, bp2, P3r, bp3)


def kernel(x, t, z, edge_index, W1, b1, W2, b2, P1, bp1, P2, bp2, P3, bp3):
    t2 = t[:, None]
    y, xcP = _pre(x, t2, W1[:D], W1[D:D + 1], P1[H:H + D], P1[H + D:H + D + 1])

    src = edge_index[0]
    dst = edge_index[1]
    pad = EPAD - E
    srcp = jnp.concatenate([src, jnp.zeros((pad,), jnp.int32)])
    dstp = jnp.concatenate([dst, jnp.full((pad,), NROWS - 1, jnp.int32)])
    src0 = srcp[:E0].reshape(NS, C0, CHUNK)
    dst0 = dstp[:E0].reshape(NS, C0, CHUNK)
    src1 = srcp[E0:].reshape(NS, C1, CHUNK)
    dst1 = dstp[E0:].reshape(NS, C1, CHUNK)

    parts = _sc_aggregate(y, src0, dst0, src1, dst1)

    p = _post(y, xcP, parts, b1[None, :], W2, b2[None, :],
              P1[:H], bp1[None, :], P2, bp2[None, :], P3.reshape(1, H), bp3[None, :])

    t_pred = jnp.zeros((N, 1), jnp.float32)
    return (t_pred, p)
